# baseline (device time: 1926788 ns/iter reference)
import jax
import jax.numpy as jnp
from jax import lax
from jax.experimental import pallas as pl
from jax.experimental.pallas import tpu as pltpu

N_DEV = 32


def _hamiltonian_cycle():
    path = []
    for y in range(4):
        zs = range(4) if y % 2 == 0 else range(3, -1, -1)
        path.extend((y, z) for z in zs)
    cyc = [(0, y, z) for (y, z) in path]
    cyc += [(1, y, z) for (y, z) in reversed(path)]

    def midx(x, y, z):
        return z * 8 + y * 2 + (x if y % 2 == 0 else 1 - x)

    ring = [midx(*c) for c in cyc]
    assert sorted(ring) == list(range(N_DEV))
    inv = [0] * N_DEV
    for p, m in enumerate(ring):
        inv[m] = p
    return ring, inv

_RING, _INV = _hamiltonian_cycle()


def _ring_allreduce(y, scal):
    M, N = y.shape
    H = M // 2
    CH = H // N_DEV

    def body(scal_ref, y_ref, out_ref,
             send_a, recv_a, send_b, recv_b,
             send_sems_a, recv_sems_a, send_sems_b, recv_sems_b,
             load_sems_a, load_sems_b, store_sem_a, store_sem_b,
             credit_a, credit_b, qstore_sems, amax_ref):
        pos = scal_ref[0]
        left = scal_ref[1]
        right = scal_ref[2]
        amax_ref[0] = jnp.float32(0.0)

        barrier_sem = pltpu.get_barrier_semaphore()
        for nbr in (left, right):
            pl.semaphore_signal(
                barrier_sem, inc=1,
                device_id=(nbr,), device_id_type=pl.DeviceIdType.MESH,
            )
        pl.semaphore_wait(barrier_sem, 2)

        def load(row0, vbuf, sem):
            cp = pltpu.make_async_copy(
                y_ref.at[pl.ds(row0, CH), :], vbuf, sem)
            cp.start()
            return cp

        def store(vbuf, row0, sem):
            cp = pltpu.make_async_copy(
                vbuf, out_ref.at[pl.ds(row0, CH), :], sem)
            cp.start()
            return cp

        def rdma_pair(slot):
            ra = pltpu.make_async_remote_copy(
                src_ref=send_a.at[slot], dst_ref=recv_a.at[slot],
                send_sem=send_sems_a.at[slot], recv_sem=recv_sems_a.at[slot],
                device_id=(right,), device_id_type=pl.DeviceIdType.MESH,
            )
            rb = pltpu.make_async_remote_copy(
                src_ref=send_b.at[slot], dst_ref=recv_b.at[slot],
                send_sem=send_sems_b.at[slot], recv_sem=recv_sems_b.at[slot],
                device_id=(left,), device_id_type=pl.DeviceIdType.MESH,
            )
            return ra, rb

        def credits(g):
            pl.semaphore_signal(
                credit_a, inc=1,
                device_id=(left,), device_id_type=pl.DeviceIdType.MESH,
            )
            pl.semaphore_signal(
                credit_b, inc=1,
                device_id=(right,), device_id_type=pl.DeviceIdType.MESH,
            )

        def wait_credits(g):
            @pl.when(g >= 2)
            def _():
                pl.semaphore_wait(credit_a, 1)
                pl.semaphore_wait(credit_b, 1)

        def rs_rows(g):
            ia = lax.rem(pos - (g + 1) + N_DEV, N_DEV)
            ib = lax.rem(pos + (g + 1), N_DEV)
            return ia * CH, H + ib * CH

        load(pos * CH, send_a.at[0], load_sems_a.at[0]).wait()
        load(H + pos * CH, send_b.at[0], load_sems_b.at[0]).wait()

        def rs_body(g, carry):
            slot = lax.rem(g, 2)
            nxt = lax.rem(g + 1, 2)
            wait_credits(g)
            ra, rb = rdma_pair(slot)
            ra.start()
            rb.start()

            @pl.when(g >= 1)
            def _():
                pa, pb = rdma_pair(nxt)
                pa.wait_send()
                pb.wait_send()

            row_a, row_b = rs_rows(g)
            cpa = load(row_a, send_a.at[nxt], load_sems_a.at[nxt])
            cpb = load(row_b, send_b.at[nxt], load_sems_b.at[nxt])
            cpa.wait()
            cpb.wait()
            ra.wait_recv()
            send_a[nxt] = send_a[nxt] + recv_a[slot]
            rb.wait_recv()
            send_b[nxt] = send_b[nxt] + recv_b[slot]
            credits(g)
            return carry

        lax.fori_loop(0, N_DEV - 1, rs_body, 0)

        own_a = lax.rem(pos + 1, N_DEV)
        own_b = lax.rem(pos - 1 + N_DEV, N_DEV)
        sta = store(send_a.at[1], own_a * CH, store_sem_a)
        stb = store(send_b.at[1], H + own_b * CH, store_sem_b)
        amax_ref[0] = jnp.maximum(
            jnp.max(jnp.abs(send_a[1])), jnp.max(jnp.abs(send_b[1])))
        sta.wait()
        stb.wait()

        def ag_body(t, carry):
            g = (N_DEV - 1) + t
            slot = lax.rem(g, 2)
            nxt = lax.rem(g + 1, 2)
            wait_credits(g)
            ra, rb = rdma_pair(slot)
            ra.start()
            rb.start()
            pa, pb = rdma_pair(nxt)
            pa.wait_send()
            pb.wait_send()
            ia = lax.rem(pos - t + N_DEV, N_DEV)
            ib = lax.rem(pos + t, N_DEV)
            ra.wait_recv()
            sta = store(recv_a.at[slot], ia * CH, store_sem_a)

            @pl.when(t < N_DEV - 2)
            def _():
                send_a[nxt] = recv_a[slot]

            rb.wait_recv()
            stb = store(recv_b.at[slot], H + ib * CH, store_sem_b)

            @pl.when(t < N_DEV - 2)
            def _():
                send_b[nxt] = recv_b[slot]

            sta.wait()
            stb.wait()
            credits(g)
            amax_ref[0] = jnp.maximum(
                amax_ref[0],
                jnp.maximum(jnp.max(jnp.abs(recv_a[slot])),
                            jnp.max(jnp.abs(recv_b[slot]))))
            return carry

        lax.fori_loop(0, N_DEV - 1, ag_body, 0)

        la, lb = rdma_pair(1)
        la.wait_send()
        lb.wait_send()
        pl.semaphore_wait(credit_a, 2)
        pl.semaphore_wait(credit_b, 2)

        amax = amax_ref[0]
        scale = amax / jnp.float32(448.0)
        inv_scale = jnp.float32(448.0) / amax

        def snap(v):
            a = jnp.abs(v) * inv_scale
            u = lax.bitcast_convert_type(a, jnp.int32)
            lsb = jnp.bitwise_and(lax.shift_right_logical(u, 20), 1)
            ur = jnp.bitwise_and(u + lsb + ((1 << 19) - 1),
                                 jnp.int32(~((1 << 20) - 1)))
            n_norm = lax.bitcast_convert_type(ur, jnp.float32)
            magic = jnp.float32(16384.0)
            n_sub = (a + magic) - magic
            snapped = jnp.where(a >= jnp.float32(2.0 ** -6), n_norm, n_sub)
            snapped = jnp.minimum(snapped, jnp.float32(448.0))
            return jnp.sign(v) * snapped * scale

        n_q = M // CH

        def qload_cp(k):
            slot = lax.rem(k, 2)
            return pltpu.make_async_copy(
                out_ref.at[pl.ds(k * CH, CH), :], recv_a.at[slot],
                load_sems_a.at[slot])

        def qstore_cp(k):
            slot = lax.rem(k, 2)
            return pltpu.make_async_copy(
                send_a.at[slot], out_ref.at[pl.ds(k * CH, CH), :],
                qstore_sems.at[slot])

        qload_cp(0).start()

        def qbody(k, carry):
            slot = lax.rem(k, 2)

            @pl.when(k + 1 < n_q)
            def _():
                qload_cp(k + 1).start()

            qload_cp(k).wait()

            @pl.when(k >= 2)
            def _():
                qstore_cp(k - 2).wait()

            send_a[slot] = snap(recv_a[slot])
            qstore_cp(k).start()
            return carry

        lax.fori_loop(0, n_q, qbody, 0)
        qstore_cp(n_q - 2).wait()
        qstore_cp(n_q - 1).wait()

    return pl.pallas_call(
        body,
        out_shape=jax.ShapeDtypeStruct((M, N), jnp.float32),
        in_specs=[
            pl.BlockSpec(memory_space=pltpu.MemorySpace.SMEM),
            pl.BlockSpec(memory_space=pl.ANY),
        ],
        out_specs=pl.BlockSpec(memory_space=pl.ANY),
        scratch_shapes=[
            pltpu.VMEM((2, CH, N), jnp.float32),
            pltpu.VMEM((2, CH, N), jnp.float32),
            pltpu.VMEM((2, CH, N), jnp.float32),
            pltpu.VMEM((2, CH, N), jnp.float32),
            pltpu.SemaphoreType.DMA((2,)),
            pltpu.SemaphoreType.DMA((2,)),
            pltpu.SemaphoreType.DMA((2,)),
            pltpu.SemaphoreType.DMA((2,)),
            pltpu.SemaphoreType.DMA((2,)),
            pltpu.SemaphoreType.DMA((2,)),
            pltpu.SemaphoreType.DMA,
            pltpu.SemaphoreType.DMA,
            pltpu.SemaphoreType.REGULAR,
            pltpu.SemaphoreType.REGULAR,
            pltpu.SemaphoreType.DMA((2,)),
            pltpu.SMEM((1,), jnp.float32),
        ],
        compiler_params=pltpu.CompilerParams(collective_id=0),
    )(scal, y)


def kernel(x, w_mat):
    y = jnp.dot(x, w_mat, preferred_element_type=jnp.float32,
                precision=lax.Precision.HIGHEST)
    r = lax.axis_index("i")
    ring = jnp.asarray(_RING, jnp.int32)
    pos = jnp.asarray(_INV, jnp.int32)[r]
    right = ring[lax.rem(pos + 1, N_DEV)]
    left = ring[lax.rem(pos - 1 + N_DEV, N_DEV)]
    scal = jnp.stack([pos, left, right]).astype(jnp.int32)
    return _ring_allreduce(y, scal)
